# baseline (speedup 1.0000x reference)
import functools

import jax
import jax.numpy as jnp
from jax import lax
from jax.experimental import pallas as pl
from jax.experimental.pallas import tpu as pltpu
from jax.experimental.pallas import tpu_sc as plsc

BATCH = 16384
D_OUT = 64
BM = 1000


def _proj_body(offs, steps, *refs):
    e_refs, w_refs, o_refs = refs[0:4], refs[4:8], refs[8:12]
    pid = pl.program_id(0)
    for j in range(4):
        @pl.when(jnp.logical_and(pid >= offs[j], pid < offs[j] + steps[j]))
        def _(j=j):
            o_refs[j][...] = jnp.dot(e_refs[j][...], w_refs[j][...],
                                     preferred_element_type=jnp.float32)


def _project_all(tables, weights):
    steps = tuple(e.shape[0] // BM for e in tables)
    offs = []
    acc = 0
    for st in steps:
        offs.append(acc)
        acc += st
    offs = tuple(offs)
    total = acc

    def e_map(i, off=0, hi=0):
        return (jnp.clip(i - off, 0, hi), 0)

    in_specs = [
        pl.BlockSpec((BM, e.shape[1]),
                     functools.partial(e_map, off=offs[j], hi=steps[j] - 1))
        for j, e in enumerate(tables)
    ] + [
        pl.BlockSpec(w.shape, lambda i: (0, 0)) for w in weights
    ]
    out_specs = [
        pl.BlockSpec((BM, D_OUT),
                     functools.partial(e_map, off=offs[j], hi=steps[j] - 1))
        for j in range(4)
    ]
    out_shape = [jax.ShapeDtypeStruct((e.shape[0], D_OUT), jnp.float32)
                 for e in tables]
    return pl.pallas_call(
        functools.partial(_proj_body, offs, steps),
        grid=(total,),
        in_specs=in_specs,
        out_specs=out_specs,
        out_shape=out_shape,
    )(*tables, *weights)


FBM = 2048
FSTEPS = BATCH // FBM


def _finish_body(xu, xi, wu, wi, bu, bi, g0u, g1u, g0i, g1i, ou, oi):
    pid = pl.program_id(0)

    @pl.when(pid < FSTEPS)
    def _():
        ou[...] = (jnp.dot(xu[...], wu[...],
                           preferred_element_type=jnp.float32)
                   + bu[...] + g0u[...] + g1u[...])

    @pl.when(pid >= FSTEPS)
    def _():
        oi[...] = (jnp.dot(xi[...], wi[...],
                           preferred_element_type=jnp.float32)
                   + bi[...] + g0i[...] + g1i[...])


def _finish_both(xu, wu, bu, g0u, g1u, xi, wi, bi, g0i, g1i):
    d_in = xu.shape[1]

    def umap(i):
        return (jnp.clip(i, 0, FSTEPS - 1), 0)

    def imap(i):
        return (jnp.clip(i - FSTEPS, 0, FSTEPS - 1), 0)

    big_u = pl.BlockSpec((FBM, D_OUT), umap)
    big_i = pl.BlockSpec((FBM, D_OUT), imap)
    const = lambda shape: pl.BlockSpec(shape, lambda i: (0, 0))
    return pl.pallas_call(
        _finish_body,
        grid=(2 * FSTEPS,),
        in_specs=[
            pl.BlockSpec((FBM, d_in), umap), pl.BlockSpec((FBM, d_in), imap),
            const((d_in, D_OUT)), const((d_in, D_OUT)),
            const((1, D_OUT)), const((1, D_OUT)),
            big_u, big_u, big_i, big_i,
        ],
        out_specs=[big_u, big_i],
        out_shape=[jax.ShapeDtypeStruct((BATCH, D_OUT), jnp.float32)] * 2,
    )(xu, xi, wu, wi, bu, bi, g0u, g1u, g0i, g1i)


@functools.lru_cache(maxsize=None)
def _sc_gather_fn():
    info = plsc.get_sparse_core_info()
    nc, ns = info.num_cores, info.num_subcores
    nw = nc * ns
    bpw = BATCH // nw

    mesh = plsc.VectorSubcoreMesh(core_axis_name="c", subcore_axis_name="s")

    def body(t0, t1, t2, t3, i0, i1, i2, i3, g0, g1, g2, g3,
             idx0_v, idx1_v, idx2_v, idx3_v, r0, r1, r2, gsem, wsem):
        wid = lax.axis_index("s") * nc + lax.axis_index("c")
        base = wid * bpw
        sl = pl.ds(base, bpw)
        pltpu.sync_copy(i0.at[sl], idx0_v)
        pltpu.sync_copy(i1.at[sl], idx1_v)
        pltpu.sync_copy(i2.at[sl], idx2_v)
        pltpu.sync_copy(i3.at[sl], idx3_v)
        d0 = pltpu.async_copy(t0.at[idx0_v], r0, gsem)
        d1 = pltpu.async_copy(t1.at[idx1_v], r1, gsem)
        d2 = pltpu.async_copy(t2.at[idx2_v], r2, gsem)
        d0.wait()
        w0 = pltpu.async_copy(r0, g0.at[sl], wsem)
        d1.wait()
        w1 = pltpu.async_copy(r1, g1.at[sl], wsem)
        w0.wait()
        d3 = pltpu.async_copy(t3.at[idx3_v], r0, gsem)
        d2.wait()
        w2 = pltpu.async_copy(r2, g2.at[sl], wsem)
        d3.wait()
        w3 = pltpu.async_copy(r0, g3.at[sl], wsem)
        w1.wait()
        w2.wait()
        w3.wait()

    out = jax.ShapeDtypeStruct((BATCH, D_OUT), jnp.float32)
    return pl.kernel(
        body,
        out_type=(out, out, out, out),
        mesh=mesh,
        scratch_types=[
            pltpu.VMEM((bpw,), jnp.int32),
            pltpu.VMEM((bpw,), jnp.int32),
            pltpu.VMEM((bpw,), jnp.int32),
            pltpu.VMEM((bpw,), jnp.int32),
            pltpu.VMEM((bpw, D_OUT), jnp.float32),
            pltpu.VMEM((bpw, D_OUT), jnp.float32),
            pltpu.VMEM((bpw, D_OUT), jnp.float32),
            pltpu.SemaphoreType.DMA,
            pltpu.SemaphoreType.DMA,
        ],
        compiler_params=pltpu.CompilerParams(use_tc_tiling_on_sc=False),
    )


def kernel(x_user, x_item, emb_user_0, emb_user_1, emb_item_0, emb_item_1,
           W_user, b_user, W_item, b_item):
    d_in = x_user.shape[1]
    d0u = emb_user_0.shape[1]
    d0i = emb_item_0.shape[1]

    idx0u = x_user[:, 0].astype(jnp.int32)
    idx1u = x_user[:, 1].astype(jnp.int32)
    idx0i = x_item[:, 0].astype(jnp.int32)
    idx1i = x_item[:, 1].astype(jnp.int32)

    t0u, t1u, t0i, t1i = _project_all(
        (emb_user_0, emb_user_1, emb_item_0, emb_item_1),
        (W_user[d_in:d_in + d0u], W_user[d_in + d0u:],
         W_item[d_in:d_in + d0i], W_item[d_in + d0i:]))

    g0u, g1u, g0i, g1i = _sc_gather_fn()(
        t0u, t1u, t0i, t1i, idx0u, idx1u, idx0i, idx1i)

    return _finish_both(
        x_user, W_user[:d_in], b_user.reshape(1, -1), g0u, g1u,
        x_item, W_item[:d_in], b_item.reshape(1, -1), g0i, g1i)
